# baseline (speedup 1.0000x reference)
import functools

import jax
import jax.numpy as jnp
from jax.experimental import pallas as pl
from jax.experimental.pallas import tpu as pltpu

_B = 800
_D = 128

_INV2PI = 0.15915494309189535
_2PI_HI = 6.283185482025146
_2PI_LO = -1.7484556000744083e-07
_C0 = 0.9999992
_C1 = -0.49999422
_C2 = 0.041659776
_C3 = -0.001385879
_C4 = 2.4202942e-05
_C5 = -2.1972964e-07


def _fast_cos(x):
    k = jnp.floor(x * _INV2PI + 0.5)
    r = (x - k * _2PI_HI) - k * _2PI_LO
    s = r * r
    return _C0 + s * (_C1 + s * (_C2 + s * (_C3 + s * (_C4 + s * _C5))))


def _tsage_kernel(
    src_ref, dst_ref, ts_ref,
    ssc_ref, ssr_ref, dsc_ref, dsr_ref,
    freq_ref, phase_ref, w1a_ref, w1b_ref, b1_ref, ws_ref, wn_ref, bo_ref,
    tri_ref, ones_ref,
    osrc_ref, odst_ref,
    rs_ref, ns_ref, ls_ref, rd_ref, nd_ref, ld_ref,
):
    i = pl.program_id(0)

    @pl.when(i == 0)
    def _init():
        rs_ref[...] = jnp.zeros_like(rs_ref)
        rd_ref[...] = jnp.zeros_like(rd_ref)
        ns_ref[...] = jnp.zeros_like(ns_ref)
        nd_ref[...] = jnp.zeros_like(nd_ref)
        ls_ref[...] = jnp.full_like(ls_ref, -1)
        ld_ref[...] = jnp.full_like(ld_ref, -1)

    f32 = jnp.float32
    bf16 = jnp.bfloat16
    t = ts_ref[...]
    tenc = _fast_cos(t * freq_ref[...] + phase_ref[...]).astype(bf16)
    tw = jnp.dot(tenc, w1b_ref[...], preferred_element_type=f32) + b1_ref[...]

    h_src = jax.nn.relu(
        jnp.dot(src_ref[...].astype(bf16), w1a_ref[...],
                preferred_element_type=f32) + tw).astype(bf16)
    h_dst = jax.nn.relu(
        jnp.dot(dst_ref[...].astype(bf16), w1a_ref[...],
                preferred_element_type=f32) + tw).astype(bf16)

    p_src = jnp.dot(h_src, wn_ref[...], preferred_element_type=f32).astype(bf16)
    p_dst = jnp.dot(h_dst, wn_ref[...], preferred_element_type=f32).astype(bf16)

    tri = tri_ref[...]
    ones = ones_ref[...]

    def seg_cummean(p, s_col, s_row, r_ref, n_ref, l_ref):
        m = jnp.where(s_col == s_row, tri, bf16(0.0))
        seg = jnp.dot(m, p, preferred_element_type=f32)
        cnt = jnp.dot(m, ones, preferred_element_type=f32)
        cont = (s_col == l_ref[...]).astype(f32)
        x_tot = seg + cont * r_ref[...]
        c_tot = cnt + cont * n_ref[...]
        r_ref[...] = x_tot[_B - 1:_B, :]
        n_ref[...] = c_tot[_B - 1:_B, 0:1]
        l_ref[...] = s_col[_B - 1:_B, :]
        return x_tot / c_tot

    mean_src = seg_cummean(p_dst, ssc_ref[...], ssr_ref[0], rs_ref, ns_ref, ls_ref)
    mean_dst = seg_cummean(p_src, dsc_ref[...], dsr_ref[0], rd_ref, nd_ref, ld_ref)

    bo = bo_ref[...]
    osrc_ref[...] = (
        jnp.dot(h_src, ws_ref[...], preferred_element_type=f32) + mean_src + bo)
    odst_ref[...] = (
        jnp.dot(h_dst, ws_ref[...], preferred_element_type=f32) + mean_dst + bo)


@jax.jit
def kernel(src_feat, dst_feat, timestamp, src_segids, dst_segids,
           basis_freq, phase, W1, b1, W_self, b_self, W_neigh, b_neigh):
    e, d = src_feat.shape
    nb = e // _B
    f32 = jnp.float32
    bf16 = jnp.bfloat16

    ts = timestamp.reshape(e, 1)
    ssc = src_segids.astype(jnp.int32).reshape(e, 1)
    dsc = dst_segids.astype(jnp.int32).reshape(e, 1)
    ssr = src_segids.astype(jnp.int32).reshape(nb, 1, _B)
    dsr = dst_segids.astype(jnp.int32).reshape(nb, 1, _B)
    w1a = W1[:, :d].T.astype(bf16)
    w1b = W1[:, d:].T.astype(bf16)
    ws = W_self.T.astype(bf16)
    wn = W_neigh.T.astype(bf16)
    bo = (b_self + b_neigh).reshape(1, d)
    b1r = b1.reshape(1, d)
    fr = basis_freq.reshape(1, d)
    ph = phase.reshape(1, d)
    tri = jnp.tril(jnp.ones((_B, _B), bf16))
    ones_bd = jnp.ones((_B, d), bf16)

    row_spec = pl.BlockSpec((_B, d), lambda i: (i, 0))
    col1_spec = pl.BlockSpec((_B, 1), lambda i: (i, 0))
    seg_row_spec = pl.BlockSpec((1, 1, _B), lambda i: (i, 0, 0))
    full2 = lambda a, b: pl.BlockSpec((a, b), lambda i: (0, 0))

    out = pl.pallas_call(
        _tsage_kernel,
        grid=(nb,),
        in_specs=[
            row_spec, row_spec, col1_spec,
            col1_spec, seg_row_spec, col1_spec, seg_row_spec,
            full2(1, d), full2(1, d), full2(d, d), full2(d, d),
            full2(1, d), full2(d, d), full2(d, d), full2(1, d),
            full2(_B, _B), full2(_B, d),
        ],
        out_specs=[row_spec, row_spec],
        out_shape=[jax.ShapeDtypeStruct((e, d), f32),
                   jax.ShapeDtypeStruct((e, d), f32)],
        scratch_shapes=[
            pltpu.VMEM((1, d), f32), pltpu.VMEM((1, 1), f32),
            pltpu.VMEM((1, 1), jnp.int32),
            pltpu.VMEM((1, d), f32), pltpu.VMEM((1, 1), f32),
            pltpu.VMEM((1, 1), jnp.int32),
        ],
        compiler_params=pltpu.CompilerParams(
            dimension_semantics=("arbitrary",)),
    )(src_feat, dst_feat, ts, ssc, ssr, dsc, dsr,
      fr, ph, w1a, w1b, b1r, ws, wn, bo, tri, ones_bd)
    return (out[0], out[1])


# speedup vs baseline: 1.1034x; 1.1034x over previous
import functools

import jax
import jax.numpy as jnp
from jax.experimental import pallas as pl
from jax.experimental.pallas import tpu as pltpu

_B = 640
_D = 128

_INV2PI = 0.15915494309189535
_2PI_HI = 6.283185482025146
_2PI_LO = -1.7484556000744083e-07
_C0 = 0.9999992
_C1 = -0.49999422
_C2 = 0.041659776
_C3 = -0.001385879
_C4 = 2.4202942e-05
_C5 = -2.1972964e-07


def _fast_cos(x):
    k = jnp.floor(x * _INV2PI + 0.5)
    r = (x - k * _2PI_HI) - k * _2PI_LO
    s = r * r
    return _C0 + s * (_C1 + s * (_C2 + s * (_C3 + s * (_C4 + s * _C5))))


def _tsage_kernel(
    src_ref, dst_ref, ts_ref,
    ssc_ref, ssr_ref, dsc_ref, dsr_ref,
    freq_ref, phase_ref, w1a_ref, w1b_ref, b1_ref, ws_ref, wn_ref, bo_ref,
    tri_ref, ones_ref,
    osrc_ref, odst_ref,
    rs_ref, ns_ref, ls_ref, rd_ref, nd_ref, ld_ref,
):
    i = pl.program_id(0)

    @pl.when(i == 0)
    def _init():
        rs_ref[...] = jnp.zeros_like(rs_ref)
        rd_ref[...] = jnp.zeros_like(rd_ref)
        ns_ref[...] = jnp.zeros_like(ns_ref)
        nd_ref[...] = jnp.zeros_like(nd_ref)
        ls_ref[...] = jnp.full_like(ls_ref, -1)
        ld_ref[...] = jnp.full_like(ld_ref, -1)

    f32 = jnp.float32
    bf16 = jnp.bfloat16
    t = ts_ref[...]
    tenc = _fast_cos(t * freq_ref[...] + phase_ref[...]).astype(bf16)
    tw = jnp.dot(tenc, w1b_ref[...], preferred_element_type=f32) + b1_ref[...]

    h_src = jax.nn.relu(
        jnp.dot(src_ref[...].astype(bf16), w1a_ref[...],
                preferred_element_type=f32) + tw).astype(bf16)
    h_dst = jax.nn.relu(
        jnp.dot(dst_ref[...].astype(bf16), w1a_ref[...],
                preferred_element_type=f32) + tw).astype(bf16)

    p_src = jnp.dot(h_src, wn_ref[...], preferred_element_type=f32).astype(bf16)
    p_dst = jnp.dot(h_dst, wn_ref[...], preferred_element_type=f32).astype(bf16)

    tri = tri_ref[...]
    ones = ones_ref[...]

    def seg_cummean(p, s_col, s_row, r_ref, n_ref, l_ref):
        m = jnp.where(s_col == s_row, tri, bf16(0.0))
        seg = jnp.dot(m, p, preferred_element_type=f32)
        cnt = jnp.dot(m, ones, preferred_element_type=f32)
        cont = (s_col == l_ref[...]).astype(f32)
        x_tot = seg + cont * r_ref[...]
        c_tot = cnt + cont * n_ref[...]
        r_ref[...] = x_tot[_B - 1:_B, :]
        n_ref[...] = c_tot[_B - 1:_B, 0:1]
        l_ref[...] = s_col[_B - 1:_B, :]
        return x_tot / c_tot

    mean_src = seg_cummean(p_dst, ssc_ref[...], ssr_ref[0], rs_ref, ns_ref, ls_ref)
    mean_dst = seg_cummean(p_src, dsc_ref[...], dsr_ref[0], rd_ref, nd_ref, ld_ref)

    bo = bo_ref[...]
    osrc_ref[...] = (
        jnp.dot(h_src, ws_ref[...], preferred_element_type=f32) + mean_src + bo)
    odst_ref[...] = (
        jnp.dot(h_dst, ws_ref[...], preferred_element_type=f32) + mean_dst + bo)


@jax.jit
def kernel(src_feat, dst_feat, timestamp, src_segids, dst_segids,
           basis_freq, phase, W1, b1, W_self, b_self, W_neigh, b_neigh):
    e, d = src_feat.shape
    nb = e // _B
    f32 = jnp.float32
    bf16 = jnp.bfloat16

    ts = timestamp.reshape(e, 1)
    ssc = src_segids.astype(jnp.int32).reshape(e, 1)
    dsc = dst_segids.astype(jnp.int32).reshape(e, 1)
    ssr = src_segids.astype(jnp.int32).reshape(nb, 1, _B)
    dsr = dst_segids.astype(jnp.int32).reshape(nb, 1, _B)
    w1a = W1[:, :d].T.astype(bf16)
    w1b = W1[:, d:].T.astype(bf16)
    ws = W_self.T.astype(bf16)
    wn = W_neigh.T.astype(bf16)
    bo = (b_self + b_neigh).reshape(1, d)
    b1r = b1.reshape(1, d)
    fr = basis_freq.reshape(1, d)
    ph = phase.reshape(1, d)
    tri = jnp.tril(jnp.ones((_B, _B), bf16))
    ones_bd = jnp.ones((_B, d), bf16)

    row_spec = pl.BlockSpec((_B, d), lambda i: (i, 0))
    col1_spec = pl.BlockSpec((_B, 1), lambda i: (i, 0))
    seg_row_spec = pl.BlockSpec((1, 1, _B), lambda i: (i, 0, 0))
    full2 = lambda a, b: pl.BlockSpec((a, b), lambda i: (0, 0))

    out = pl.pallas_call(
        _tsage_kernel,
        grid=(nb,),
        in_specs=[
            row_spec, row_spec, col1_spec,
            col1_spec, seg_row_spec, col1_spec, seg_row_spec,
            full2(1, d), full2(1, d), full2(d, d), full2(d, d),
            full2(1, d), full2(d, d), full2(d, d), full2(1, d),
            full2(_B, _B), full2(_B, d),
        ],
        out_specs=[row_spec, row_spec],
        out_shape=[jax.ShapeDtypeStruct((e, d), f32),
                   jax.ShapeDtypeStruct((e, d), f32)],
        scratch_shapes=[
            pltpu.VMEM((1, d), f32), pltpu.VMEM((1, 1), f32),
            pltpu.VMEM((1, 1), jnp.int32),
            pltpu.VMEM((1, d), f32), pltpu.VMEM((1, 1), f32),
            pltpu.VMEM((1, 1), jnp.int32),
        ],
        compiler_params=pltpu.CompilerParams(
            dimension_semantics=("parallel",)),
    )(src_feat, dst_feat, ts, ssc, ssr, dsc, dsr,
      fr, ph, w1a, w1b, b1r, ws, wn, bo, tri, ones_bd)
    return (out[0], out[1])
